# Initial kernel scaffold; baseline (speedup 1.0000x reference)
import jax
import jax.numpy as jnp
from jax.experimental import pallas as pl


def kernel(x, w11, b11, w12, b12, w21, b21, w22, b22, wfc, bfc):
    raise NotImplementedError("write your pallas kernel here")


# speedup vs baseline: 1.7884x; 1.7884x over previous
import jax
import jax.numpy as jnp
from jax.experimental import pallas as pl
from jax.experimental.pallas import tpu as pltpu

_BF = jnp.bfloat16


def _shift_cat(y4, W):
    B, H, _, C = y4.shape
    wi = jax.lax.broadcasted_iota(jnp.int32, y4.shape, 2)
    zero = jnp.zeros((), y4.dtype)
    left = jnp.where(wi >= 1, pltpu.roll(y4, shift=1, axis=2), zero)
    right = jnp.where(wi <= W - 2, pltpu.roll(y4, shift=W - 1, axis=2), zero)
    return jnp.concatenate([left, y4, right], axis=3).reshape(B * H * W, 3 * C)


def _conv3x3_relu(xcat, wmat, bias, B, H, W, Cout):
    g = jnp.dot(xcat, wmat, preferred_element_type=jnp.float32)
    g4 = g.reshape(B, H, W, 3 * Cout)
    g0 = jax.lax.slice_in_dim(g4, 0, Cout, axis=3)
    g1 = jax.lax.slice_in_dim(g4, Cout, 2 * Cout, axis=3)
    g2 = jax.lax.slice_in_dim(g4, 2 * Cout, 3 * Cout, axis=3)
    z = jnp.zeros((B, 1, W, Cout), jnp.float32)
    acc = (g1
           + jnp.concatenate([z, g0[:, :H - 1]], axis=1)
           + jnp.concatenate([g2[:, 1:], z], axis=1))
    return jnp.maximum(acc + bias.reshape(1, 1, 1, Cout), 0.0)


def _maxpool2(y4):
    B, H, W, C = y4.shape
    r = y4.reshape(B, H // 2, 2, W // 2, 2, C)
    return jnp.max(r, axis=(2, 4))


def _body(x_ref, w1_ref, b1_ref, w2_ref, b2_ref, w3_ref, b3_ref,
          w4_ref, b4_ref, wfc_ref, bfc_ref, out_ref):
    B = x_ref.shape[0]

    xc = _shift_cat(x_ref[...], 32)
    y = _conv3x3_relu(xc, w1_ref[...], b1_ref[...], B, 32, 32, 32)
    xc = _shift_cat(y.astype(_BF), 32)
    y = _conv3x3_relu(xc, w2_ref[...], b2_ref[...], B, 32, 32, 32)
    p = _maxpool2(y)
    xc = _shift_cat(p.astype(_BF), 16)
    y = _conv3x3_relu(xc, w3_ref[...], b3_ref[...], B, 16, 16, 64)
    xc = _shift_cat(y.astype(_BF), 16)
    y = _conv3x3_relu(xc, w4_ref[...], b4_ref[...], B, 16, 16, 64)
    p = _maxpool2(y)

    pf = p.reshape(B, 8 * 8 * 64).astype(_BF)
    logits = jnp.dot(pf, wfc_ref[...], preferred_element_type=jnp.float32)
    out_ref[...] = logits + bfc_ref[...]


def _prep_conv_w(w_oihw):
    cout, cin = w_oihw.shape[0], w_oihw.shape[1]
    return (jnp.transpose(w_oihw, (3, 1, 2, 0))
            .reshape(3 * cin, 3 * cout).astype(_BF))


def kernel(x, w11, b11, w12, b12, w21, b21, w22, b22, wfc, bfc):
    N, C, H, W = x.shape
    nc = wfc.shape[0]
    B = 16
    n_pad = (-N) % B

    x_nhwc = jnp.transpose(x, (0, 2, 3, 1)).astype(_BF)
    if n_pad:
        x_nhwc = jnp.pad(x_nhwc, ((0, n_pad), (0, 0), (0, 0), (0, 0)))
    Np = N + n_pad

    w1 = _prep_conv_w(w11)
    w2 = _prep_conv_w(w12)
    w3 = _prep_conv_w(w21)
    w4 = _prep_conv_w(w22)
    wfc_k = (wfc.reshape(nc, 64, 8, 8)
             .transpose(2, 3, 1, 0)
             .reshape(64 * 64, nc).astype(_BF))

    out = pl.pallas_call(
        _body,
        out_shape=jax.ShapeDtypeStruct((Np, nc), jnp.float32),
        grid=(Np // B,),
        in_specs=[
            pl.BlockSpec((B, 32, 32, 3), lambda n: (n, 0, 0, 0)),
            pl.BlockSpec((9, 96), lambda n: (0, 0)),
            pl.BlockSpec((1, 32), lambda n: (0, 0)),
            pl.BlockSpec((96, 96), lambda n: (0, 0)),
            pl.BlockSpec((1, 32), lambda n: (0, 0)),
            pl.BlockSpec((96, 192), lambda n: (0, 0)),
            pl.BlockSpec((1, 64), lambda n: (0, 0)),
            pl.BlockSpec((192, 192), lambda n: (0, 0)),
            pl.BlockSpec((1, 64), lambda n: (0, 0)),
            pl.BlockSpec((64 * 64, nc), lambda n: (0, 0)),
            pl.BlockSpec((1, nc), lambda n: (0, 0)),
        ],
        out_specs=pl.BlockSpec((B, nc), lambda n: (n, 0)),
        compiler_params=pltpu.CompilerParams(
            dimension_semantics=("parallel",),
            vmem_limit_bytes=64 * 1024 * 1024,
        ),
    )(x_nhwc, w1, b11.reshape(1, -1), w2, b12.reshape(1, -1),
      w3, b21.reshape(1, -1), w4, b22.reshape(1, -1), wfc_k,
      bfc.reshape(1, -1))
    return out[:N]


# speedup vs baseline: 2.1751x; 1.2162x over previous
import jax
import jax.numpy as jnp
from jax.experimental import pallas as pl
from jax.experimental.pallas import tpu as pltpu

_BF = jnp.bfloat16


def _shift_cat(y4, W):
    B, H, _, C = y4.shape
    wi = jax.lax.broadcasted_iota(jnp.int32, y4.shape, 2)
    zero = jnp.zeros((), y4.dtype)
    left = jnp.where(wi >= 1, pltpu.roll(y4, shift=1, axis=2), zero)
    right = jnp.where(wi <= W - 2, pltpu.roll(y4, shift=W - 1, axis=2), zero)
    return jnp.concatenate([left, y4, right], axis=3).reshape(B * H * W, 3 * C)


def _conv3x3(xcat, wmat, B, H, W, Cout):
    g = jnp.dot(xcat, wmat, preferred_element_type=jnp.float32)
    g4 = g.reshape(B, H, W, 3 * Cout)
    g0 = jax.lax.slice_in_dim(g4, 0, Cout, axis=3)
    g1 = jax.lax.slice_in_dim(g4, Cout, 2 * Cout, axis=3)
    g2 = jax.lax.slice_in_dim(g4, 2 * Cout, 3 * Cout, axis=3)
    z = jnp.zeros((B, 1, W, Cout), jnp.float32)
    return (g1
            + jnp.concatenate([z, g0[:, :H - 1]], axis=1)
            + jnp.concatenate([g2[:, 1:], z], axis=1))


def _bias_relu(acc, bias):
    C = acc.shape[-1]
    return jnp.maximum(acc + bias.reshape(1, 1, 1, C), 0.0)


def _maxpool2(y4, mp_ref):
    B, H, W, C = y4.shape
    H2, W2 = H // 2, W // 2
    r = y4.reshape(B, H2, 2, W, C)
    mh = jnp.maximum(r[:, :, 0], r[:, :, 1])
    mw = jnp.maximum(mh, pltpu.roll(mh, shift=W - 1, axis=2))
    mp_ref[...] = mw.reshape(B * H2 * W, C)
    pooled = mp_ref[pl.ds(0, B * H2 * W2, stride=2), :]
    return pooled.reshape(B, H2, W2, C)


def _body(x_ref, w1_ref, b1_ref, w2_ref, b2_ref, w3_ref, b3_ref,
          w4_ref, b4_ref, wfc_ref, bfc_ref, out_ref, mp1_ref, mp2_ref):
    B = x_ref.shape[0]

    xc = _shift_cat(x_ref[...], 32)
    y = _bias_relu(_conv3x3(xc, w1_ref[...], B, 32, 32, 32), b1_ref[...])
    xc = _shift_cat(y.astype(_BF), 32)
    y = _conv3x3(xc, w2_ref[...], B, 32, 32, 32)
    p = _bias_relu(_maxpool2(y, mp1_ref), b2_ref[...])
    xc = _shift_cat(p.astype(_BF), 16)
    y = _bias_relu(_conv3x3(xc, w3_ref[...], B, 16, 16, 64), b3_ref[...])
    xc = _shift_cat(y.astype(_BF), 16)
    y = _conv3x3(xc, w4_ref[...], B, 16, 16, 64)
    p = _bias_relu(_maxpool2(y, mp2_ref), b4_ref[...])

    pf = p.reshape(B, 8 * 8 * 64).astype(_BF)
    logits = jnp.dot(pf, wfc_ref[...], preferred_element_type=jnp.float32)
    out_ref[...] = logits + bfc_ref[...]


def _prep_conv_w(w_oihw):
    cout, cin = w_oihw.shape[0], w_oihw.shape[1]
    return (jnp.transpose(w_oihw, (3, 1, 2, 0))
            .reshape(3 * cin, 3 * cout).astype(_BF))


def kernel(x, w11, b11, w12, b12, w21, b21, w22, b22, wfc, bfc):
    N, C, H, W = x.shape
    nc = wfc.shape[0]
    B = 16
    n_pad = (-N) % B

    x_nhwc = jnp.transpose(x, (0, 2, 3, 1)).astype(_BF)
    if n_pad:
        x_nhwc = jnp.pad(x_nhwc, ((0, n_pad), (0, 0), (0, 0), (0, 0)))
    Np = N + n_pad

    w1 = _prep_conv_w(w11)
    w2 = _prep_conv_w(w12)
    w3 = _prep_conv_w(w21)
    w4 = _prep_conv_w(w22)
    wfc_k = (wfc.reshape(nc, 64, 8, 8)
             .transpose(2, 3, 1, 0)
             .reshape(64 * 64, nc).astype(_BF))

    out = pl.pallas_call(
        _body,
        out_shape=jax.ShapeDtypeStruct((Np, nc), jnp.float32),
        grid=(Np // B,),
        in_specs=[
            pl.BlockSpec((B, 32, 32, 3), lambda n: (n, 0, 0, 0)),
            pl.BlockSpec((9, 96), lambda n: (0, 0)),
            pl.BlockSpec((1, 32), lambda n: (0, 0)),
            pl.BlockSpec((96, 96), lambda n: (0, 0)),
            pl.BlockSpec((1, 32), lambda n: (0, 0)),
            pl.BlockSpec((96, 192), lambda n: (0, 0)),
            pl.BlockSpec((1, 64), lambda n: (0, 0)),
            pl.BlockSpec((192, 192), lambda n: (0, 0)),
            pl.BlockSpec((1, 64), lambda n: (0, 0)),
            pl.BlockSpec((64 * 64, nc), lambda n: (0, 0)),
            pl.BlockSpec((1, nc), lambda n: (0, 0)),
        ],
        out_specs=pl.BlockSpec((B, nc), lambda n: (n, 0)),
        scratch_shapes=[
            pltpu.VMEM((B * 16 * 32, 32), jnp.float32),
            pltpu.VMEM((B * 8 * 16, 64), jnp.float32),
        ],
        compiler_params=pltpu.CompilerParams(
            dimension_semantics=("parallel",),
            vmem_limit_bytes=64 * 1024 * 1024,
        ),
    )(x_nhwc, w1, b11.reshape(1, -1), w2, b12.reshape(1, -1),
      w3, b21.reshape(1, -1), w4, b22.reshape(1, -1), wfc_k,
      bfc.reshape(1, -1))
    return out[:N]


# speedup vs baseline: 2.7332x; 1.2566x over previous
import jax
import jax.numpy as jnp
from jax.experimental import pallas as pl
from jax.experimental.pallas import tpu as pltpu

_BF = jnp.bfloat16


def _shift_cat(y4, W):
    B, H, _, C = y4.shape
    wi = jax.lax.broadcasted_iota(jnp.int32, y4.shape, 2)
    zero = jnp.zeros((), y4.dtype)
    left = jnp.where(wi >= 1, pltpu.roll(y4, shift=1, axis=2), zero)
    right = jnp.where(wi <= W - 2, pltpu.roll(y4, shift=W - 1, axis=2), zero)
    return jnp.concatenate([left, y4, right], axis=3).reshape(B * H * W, 3 * C)


def _conv3x3(xcat, wmat, B, H, W, Cout):
    g = jnp.dot(xcat, wmat, preferred_element_type=jnp.float32)
    g4 = g.reshape(B, H, W, 3 * Cout)
    g0 = jax.lax.slice_in_dim(g4, 0, Cout, axis=3)
    g1 = jax.lax.slice_in_dim(g4, Cout, 2 * Cout, axis=3)
    g2 = jax.lax.slice_in_dim(g4, 2 * Cout, 3 * Cout, axis=3)
    z = jnp.zeros((B, 1, W, Cout), jnp.float32)
    return (g1
            + jnp.concatenate([z, g0[:, :H - 1]], axis=1)
            + jnp.concatenate([g2[:, 1:], z], axis=1))


def _bias_relu(acc, bias):
    C = acc.shape[-1]
    return jnp.maximum(acc + bias.reshape(1, 1, 1, C), 0.0)


def _maxpool2(y4, mp_ref):
    B, H, W, C = y4.shape
    H2, W2 = H // 2, W // 2
    r = y4.reshape(B, H2, 2, W, C)
    mh = jnp.maximum(r[:, :, 0], r[:, :, 1])
    mw = jnp.maximum(mh, pltpu.roll(mh, shift=W - 1, axis=2))
    mp_ref[...] = mw.reshape(B * H2 * W, C)
    pooled = mp_ref[pl.ds(0, B * H2 * W2, stride=2), :]
    return pooled.reshape(B, H2, W2, C)


def _body(x_ref, w1_ref, b1_ref, w2_ref, b2_ref, w3_ref, b3_ref,
          w4_ref, b4_ref, wfc_ref, bfc_ref, out_ref, mp1_ref, mp2_ref):
    B = x_ref.shape[0]

    x4 = (jnp.transpose(x_ref[...].reshape(B, 3, 32 * 32), (0, 2, 1))
          .reshape(B, 32, 32, 3).astype(_BF))
    xc = _shift_cat(x4, 32)
    y = _bias_relu(_conv3x3(xc, w1_ref[...], B, 32, 32, 32), b1_ref[...])
    xc = _shift_cat(y.astype(_BF), 32)
    y = _conv3x3(xc, w2_ref[...], B, 32, 32, 32)
    p = _bias_relu(_maxpool2(y, mp1_ref), b2_ref[...])
    xc = _shift_cat(p.astype(_BF), 16)
    y = _bias_relu(_conv3x3(xc, w3_ref[...], B, 16, 16, 64), b3_ref[...])
    xc = _shift_cat(y.astype(_BF), 16)
    y = _conv3x3(xc, w4_ref[...], B, 16, 16, 64)
    p = _bias_relu(_maxpool2(y, mp2_ref), b4_ref[...])

    pf = p.reshape(B, 8 * 8 * 64).astype(_BF)
    logits = jnp.dot(pf, wfc_ref[...], preferred_element_type=jnp.float32)
    out_ref[...] = logits + bfc_ref[...]


def _prep_conv_w(w_oihw):
    cout, cin = w_oihw.shape[0], w_oihw.shape[1]
    return (jnp.transpose(w_oihw, (3, 1, 2, 0))
            .reshape(3 * cin, 3 * cout).astype(_BF))


def kernel(x, w11, b11, w12, b12, w21, b21, w22, b22, wfc, bfc):
    N, C, H, W = x.shape
    nc = wfc.shape[0]
    B = 16
    n_pad = (-N) % B

    x_in = x
    if n_pad:
        x_in = jnp.pad(x_in, ((0, n_pad), (0, 0), (0, 0), (0, 0)))
    Np = N + n_pad

    w1 = _prep_conv_w(w11)
    w2 = _prep_conv_w(w12)
    w3 = _prep_conv_w(w21)
    w4 = _prep_conv_w(w22)
    wfc_k = (wfc.reshape(nc, 64, 8, 8)
             .transpose(2, 3, 1, 0)
             .reshape(64 * 64, nc).astype(_BF))

    out = pl.pallas_call(
        _body,
        out_shape=jax.ShapeDtypeStruct((Np, nc), jnp.float32),
        grid=(Np // B,),
        in_specs=[
            pl.BlockSpec((B, 3, 32, 32), lambda n: (n, 0, 0, 0)),
            pl.BlockSpec((9, 96), lambda n: (0, 0)),
            pl.BlockSpec((1, 32), lambda n: (0, 0)),
            pl.BlockSpec((96, 96), lambda n: (0, 0)),
            pl.BlockSpec((1, 32), lambda n: (0, 0)),
            pl.BlockSpec((96, 192), lambda n: (0, 0)),
            pl.BlockSpec((1, 64), lambda n: (0, 0)),
            pl.BlockSpec((192, 192), lambda n: (0, 0)),
            pl.BlockSpec((1, 64), lambda n: (0, 0)),
            pl.BlockSpec((64 * 64, nc), lambda n: (0, 0)),
            pl.BlockSpec((1, nc), lambda n: (0, 0)),
        ],
        out_specs=pl.BlockSpec((B, nc), lambda n: (n, 0)),
        scratch_shapes=[
            pltpu.VMEM((B * 16 * 32, 32), jnp.float32),
            pltpu.VMEM((B * 8 * 16, 64), jnp.float32),
        ],
        compiler_params=pltpu.CompilerParams(
            dimension_semantics=("parallel",),
            vmem_limit_bytes=64 * 1024 * 1024,
        ),
    )(x_in, w1, b11.reshape(1, -1), w2, b12.reshape(1, -1),
      w3, b21.reshape(1, -1), w4, b22.reshape(1, -1), wfc_k,
      bfc.reshape(1, -1))
    return out[:N]


# speedup vs baseline: 4.0464x; 1.4805x over previous
import jax
import jax.numpy as jnp
from jax.experimental import pallas as pl
from jax.experimental.pallas import tpu as pltpu

_BF = jnp.bfloat16


def _shift_cat(y4, W):
    B, H, _, C = y4.shape
    wi = jax.lax.broadcasted_iota(jnp.int32, y4.shape, 2)
    zero = jnp.zeros((), y4.dtype)
    left = jnp.where(wi >= 1, pltpu.roll(y4, shift=1, axis=2), zero)
    right = jnp.where(wi <= W - 2, pltpu.roll(y4, shift=W - 1, axis=2), zero)
    return jnp.concatenate([left, y4, right], axis=3).reshape(B * H * W, 3 * C)


def _conv3x3(xcat, wmat, B, H, W, Cout):
    g = jnp.dot(xcat, wmat, preferred_element_type=jnp.float32)
    g4 = g.reshape(B, H, W, 3 * 128)
    g0 = jax.lax.slice_in_dim(g4, 0, 128, axis=3)
    g1 = jax.lax.slice_in_dim(g4, 128, 256, axis=3)
    g2 = jax.lax.slice_in_dim(g4, 256, 384, axis=3)
    z = jnp.zeros((B, 1, W, 128), jnp.float32)
    acc = (g1
           + jnp.concatenate([z, g0[:, :H - 1]], axis=1)
           + jnp.concatenate([g2[:, 1:], z], axis=1))
    return jax.lax.slice_in_dim(acc, 0, Cout, axis=3)


def _bias_relu(acc, bias):
    C = acc.shape[-1]
    return jnp.maximum(acc + bias.reshape(1, 1, 1, C), 0.0)


def _maxpool2(y4, mp_ref):
    B, H, W, C = y4.shape
    H2, W2 = H // 2, W // 2
    r = y4.reshape(B, H2, 2, W, C)
    mh = jnp.maximum(r[:, :, 0], r[:, :, 1])
    mw = jnp.maximum(mh, pltpu.roll(mh, shift=W - 1, axis=2))
    mp_ref[...] = mw.reshape(B * H2 * W, C)
    pooled = mp_ref[pl.ds(0, B * H2 * W2, stride=2), :]
    return pooled.reshape(B, H2, W2, C)


def _body(x_ref, w1_ref, b1_ref, w2_ref, b2_ref, w3_ref, b3_ref,
          w4_ref, b4_ref, wfc_ref, bfc_ref, out_ref, mp1_ref, mp2_ref):
    B = x_ref.shape[0]

    x4 = (jnp.transpose(x_ref[...].reshape(B, 3, 32 * 32), (0, 2, 1))
          .reshape(B, 32, 32, 3).astype(_BF))
    xc = _shift_cat(x4, 32)
    y = _bias_relu(_conv3x3(xc, w1_ref[...], B, 32, 32, 32), b1_ref[...])
    xc = _shift_cat(y.astype(_BF), 32)
    y = _conv3x3(xc, w2_ref[...], B, 32, 32, 32)
    p = _bias_relu(_maxpool2(y, mp1_ref), b2_ref[...])
    xc = _shift_cat(p.astype(_BF), 16)
    y = _bias_relu(_conv3x3(xc, w3_ref[...], B, 16, 16, 64), b3_ref[...])
    xc = _shift_cat(y.astype(_BF), 16)
    y = _conv3x3(xc, w4_ref[...], B, 16, 16, 64)
    p = _bias_relu(_maxpool2(y, mp2_ref), b4_ref[...])

    pf = p.reshape(B, 8 * 8 * 64).astype(_BF)
    logits = jnp.dot(pf, wfc_ref[...], preferred_element_type=jnp.float32)
    out_ref[...] = logits + bfc_ref[...]


def _prep_conv_w(w_oihw):
    cout, cin = w_oihw.shape[0], w_oihw.shape[1]
    w = jnp.transpose(w_oihw, (3, 1, 2, 0))
    w = jnp.pad(w, ((0, 0), (0, 0), (0, 0), (0, 128 - cout)))
    return w.reshape(3 * cin, 3 * 128).astype(_BF)


def kernel(x, w11, b11, w12, b12, w21, b21, w22, b22, wfc, bfc):
    N, C, H, W = x.shape
    nc = wfc.shape[0]
    B = 16
    n_pad = (-N) % B

    x_in = x
    if n_pad:
        x_in = jnp.pad(x_in, ((0, n_pad), (0, 0), (0, 0), (0, 0)))
    Np = N + n_pad

    w1 = _prep_conv_w(w11)
    w2 = _prep_conv_w(w12)
    w3 = _prep_conv_w(w21)
    w4 = _prep_conv_w(w22)
    wfc_k = (wfc.reshape(nc, 64, 8, 8)
             .transpose(2, 3, 1, 0)
             .reshape(64 * 64, nc).astype(_BF))

    out = pl.pallas_call(
        _body,
        out_shape=jax.ShapeDtypeStruct((Np, nc), jnp.float32),
        grid=(Np // B,),
        in_specs=[
            pl.BlockSpec((B, 3, 32, 32), lambda n: (n, 0, 0, 0)),
            pl.BlockSpec((9, 384), lambda n: (0, 0)),
            pl.BlockSpec((1, 32), lambda n: (0, 0)),
            pl.BlockSpec((96, 384), lambda n: (0, 0)),
            pl.BlockSpec((1, 32), lambda n: (0, 0)),
            pl.BlockSpec((96, 384), lambda n: (0, 0)),
            pl.BlockSpec((1, 64), lambda n: (0, 0)),
            pl.BlockSpec((192, 384), lambda n: (0, 0)),
            pl.BlockSpec((1, 64), lambda n: (0, 0)),
            pl.BlockSpec((64 * 64, nc), lambda n: (0, 0)),
            pl.BlockSpec((1, nc), lambda n: (0, 0)),
        ],
        out_specs=pl.BlockSpec((B, nc), lambda n: (n, 0)),
        scratch_shapes=[
            pltpu.VMEM((B * 16 * 32, 32), jnp.float32),
            pltpu.VMEM((B * 8 * 16, 64), jnp.float32),
        ],
        compiler_params=pltpu.CompilerParams(
            dimension_semantics=("parallel",),
            vmem_limit_bytes=64 * 1024 * 1024,
        ),
    )(x_in, w1, b11.reshape(1, -1), w2, b12.reshape(1, -1),
      w3, b21.reshape(1, -1), w4, b22.reshape(1, -1), wfc_k,
      bfc.reshape(1, -1))
    return out[:N]


# speedup vs baseline: 5.1398x; 1.2702x over previous
import jax
import jax.numpy as jnp
from jax.experimental import pallas as pl
from jax.experimental.pallas import tpu as pltpu

_BF = jnp.bfloat16


def _conv_dense(xd, wmat, B, H, WC):
    g = jnp.dot(xd, wmat, preferred_element_type=jnp.float32)
    g3 = g.reshape(B, H, 3 * WC)
    g0 = jax.lax.slice_in_dim(g3, 0, WC, axis=2)
    g1 = jax.lax.slice_in_dim(g3, WC, 2 * WC, axis=2)
    g2 = jax.lax.slice_in_dim(g3, 2 * WC, 3 * WC, axis=2)
    z = jnp.zeros((B, 1, WC), jnp.float32)
    return (g1
            + jnp.concatenate([z, g0[:, :H - 1]], axis=1)
            + jnp.concatenate([g2[:, 1:], z], axis=1))


def _bias_relu(acc, bias_tiled):
    return jnp.maximum(acc + bias_tiled, 0.0)


def _maxpool_dense(y3, C):
    B, H, WC = y3.shape
    r = y3.reshape(B, H // 2, 2, WC)
    mh = jnp.maximum(r[:, :, 0], r[:, :, 1]).reshape(B * (H // 2), WC)
    return jnp.maximum(mh, pltpu.roll(mh, shift=WC - C, axis=1))


def _body(x_ref, w1_ref, b1_ref, w2_ref, b2_ref, w3_ref, b3_ref,
          w4_ref, b4_ref, wfc_ref, rfc_ref, bfc_ref, out_ref):
    B = x_ref.shape[0]
    nc = out_ref.shape[1]

    xb = x_ref[...]
    xd = jnp.concatenate([xb[:, 0], xb[:, 1], xb[:, 2]], axis=2)
    xd = xd.reshape(B * 32, 96).astype(_BF)

    y = _conv_dense(xd, w1_ref[...], B, 32, 1024)
    y = _bias_relu(y, b1_ref[...].reshape(1, 1, 1024))
    y = _conv_dense(y.reshape(B * 32, 1024).astype(_BF), w2_ref[...], B, 32, 1024)
    p = _maxpool_dense(y, 32)
    p = _bias_relu(p, b2_ref[...])
    y = _conv_dense(p.astype(_BF), w3_ref[...], B, 16, 1024)
    y = _bias_relu(y, b3_ref[...].reshape(1, 1, 1024))
    y = _conv_dense(y.reshape(B * 16, 1024).astype(_BF), w4_ref[...], B, 16, 1024)
    p = _maxpool_dense(y, 64)
    p = _bias_relu(p, b4_ref[...])

    t = jnp.dot(p.astype(_BF), wfc_ref[...],
                preferred_element_type=jnp.float32)
    t4 = t.reshape(B, 8, 8 * nc)
    hi = jax.lax.broadcasted_iota(jnp.int32, t4.shape, 1)
    li = jax.lax.broadcasted_iota(jnp.int32, t4.shape, 2)
    masked = jnp.where(li // nc == hi, t4, 0.0)
    s = jnp.sum(masked, axis=1)
    logits = jnp.dot(s, rfc_ref[...], preferred_element_type=jnp.float32)
    out_ref[...] = logits + bfc_ref[...]


def _toeplitz(w_oihw, e_of_dx, c_major=False):
    groups = []
    for dy in range(3):
        t = 0.0
        for dx in range(3):
            tap = w_oihw[:, :, dy, dx].T
            e = e_of_dx(dx)
            if c_major:
                blk = jnp.einsum('co,vw->cvwo', tap, e)
                blk = blk.reshape(tap.shape[0] * e.shape[0], -1)
            else:
                blk = jnp.einsum('co,vw->vcwo', tap, e)
                blk = blk.reshape(e.shape[0] * tap.shape[0], -1)
            t = t + blk
        groups.append(t)
    return jnp.concatenate(groups, axis=1).astype(_BF)


def _eye_sel(w, dx):
    return jnp.eye(w, k=1 - dx, dtype=jnp.float32)


def _pooled_sel(win, wout, dx):
    v = jnp.arange(win)[:, None]
    w = jnp.arange(wout)[None, :]
    return (v == 2 * (w + dx - 1)).astype(jnp.float32)


def kernel(x, w11, b11, w12, b12, w21, b21, w22, b22, wfc, bfc):
    N = x.shape[0]
    nc = wfc.shape[0]
    B = 16
    n_pad = (-N) % B

    x_in = x
    if n_pad:
        x_in = jnp.pad(x_in, ((0, n_pad), (0, 0), (0, 0), (0, 0)))
    Np = N + n_pad

    w1 = _toeplitz(w11, lambda dx: _eye_sel(32, dx), c_major=True)
    w2 = _toeplitz(w12, lambda dx: _eye_sel(32, dx))
    w3 = _toeplitz(w21, lambda dx: _pooled_sel(32, 16, dx))
    w4 = _toeplitz(w22, lambda dx: _eye_sel(16, dx))
    b1t = jnp.tile(b11, 32).reshape(1, 1024)
    b2t = jnp.tile(b12, 32).reshape(1, 1024)
    b3t = jnp.tile(b21, 16).reshape(1, 1024)
    b4t = jnp.tile(b22, 16).reshape(1, 1024)
    base = (wfc.reshape(nc, 64, 8, 8)
            .transpose(3, 1, 2, 0))
    wfc_k = (jnp.stack([base, jnp.zeros_like(base)], axis=1)
             .reshape(16 * 64, 8 * nc).astype(_BF))
    rfc = jnp.tile(jnp.eye(nc, dtype=jnp.float32), (8, 1))

    out = pl.pallas_call(
        _body,
        out_shape=jax.ShapeDtypeStruct((Np, nc), jnp.float32),
        grid=(Np // B,),
        in_specs=[
            pl.BlockSpec((B, 3, 32, 32), lambda n: (n, 0, 0, 0)),
            pl.BlockSpec((96, 3072), lambda n: (0, 0)),
            pl.BlockSpec((1, 1024), lambda n: (0, 0)),
            pl.BlockSpec((1024, 3072), lambda n: (0, 0)),
            pl.BlockSpec((1, 1024), lambda n: (0, 0)),
            pl.BlockSpec((1024, 3072), lambda n: (0, 0)),
            pl.BlockSpec((1, 1024), lambda n: (0, 0)),
            pl.BlockSpec((1024, 3072), lambda n: (0, 0)),
            pl.BlockSpec((1, 1024), lambda n: (0, 0)),
            pl.BlockSpec((1024, 8 * nc), lambda n: (0, 0)),
            pl.BlockSpec((8 * nc, nc), lambda n: (0, 0)),
            pl.BlockSpec((1, nc), lambda n: (0, 0)),
        ],
        out_specs=pl.BlockSpec((B, nc), lambda n: (n, 0)),
        compiler_params=pltpu.CompilerParams(
            dimension_semantics=("parallel",),
            vmem_limit_bytes=64 * 1024 * 1024,
        ),
    )(x_in, w1, b1t, w2, b2t, w3, b3t, w4, b4t, wfc_k, rfc,
      bfc.reshape(1, -1))
    return out[:N]


# speedup vs baseline: 5.3435x; 1.0396x over previous
import jax
import jax.numpy as jnp
from jax.experimental import pallas as pl
from jax.experimental.pallas import tpu as pltpu

_BF = jnp.bfloat16


def _conv_dense(xd, wmat, B, H, WC):
    g = jnp.dot(xd, wmat, preferred_element_type=jnp.float32)
    g3 = g.reshape(B, H, 3 * WC)
    g0 = jax.lax.slice_in_dim(g3, 0, WC, axis=2)
    g1 = jax.lax.slice_in_dim(g3, WC, 2 * WC, axis=2)
    g2 = jax.lax.slice_in_dim(g3, 2 * WC, 3 * WC, axis=2)
    z = jnp.zeros((B, 1, WC), jnp.float32)
    return (g1
            + jnp.concatenate([z, g0[:, :H - 1]], axis=1)
            + jnp.concatenate([g2[:, 1:], z], axis=1))


def _bias_relu(acc, bias_tiled):
    return jnp.maximum(acc + bias_tiled, 0.0)


def _maxpool_dense(y3, C):
    B, H, WC = y3.shape
    r = y3.reshape(B, H // 2, 2, WC)
    mh = jnp.maximum(r[:, :, 0], r[:, :, 1]).reshape(B * (H // 2), WC)
    return jnp.maximum(mh, pltpu.roll(mh, shift=WC - C, axis=1))


def _body(x_ref, w1_ref, b1_ref, w2_ref, b2_ref, w3_ref, b3_ref,
          w4_ref, b4_ref, wfc_ref, rfc_ref, bfc_ref, out_ref):
    B = x_ref.shape[0]
    nc = out_ref.shape[1]

    xb = x_ref[...]
    xd = jnp.concatenate([xb[:, 0], xb[:, 1], xb[:, 2]], axis=2)
    xd = xd.reshape(B * 32, 96).astype(_BF)

    y = _conv_dense(xd, w1_ref[...], B, 32, 1024)
    y = _bias_relu(y, b1_ref[...].reshape(1, 1, 1024))
    y = _conv_dense(y.reshape(B * 32, 1024).astype(_BF), w2_ref[...], B, 32, 1024)
    p = _maxpool_dense(y, 32)
    p = _bias_relu(p, b2_ref[...])
    y = _conv_dense(p.astype(_BF), w3_ref[...], B, 16, 1024)
    y = _bias_relu(y, b3_ref[...].reshape(1, 1, 1024))
    y = _conv_dense(y.reshape(B * 16, 1024).astype(_BF), w4_ref[...], B, 16, 1024)
    p = _maxpool_dense(y, 64)
    p = _bias_relu(p, b4_ref[...])

    t = jnp.dot(p.astype(_BF), wfc_ref[...],
                preferred_element_type=jnp.float32)
    t4 = t.reshape(B, 8, 8 * nc)
    hi = jax.lax.broadcasted_iota(jnp.int32, t4.shape, 1)
    li = jax.lax.broadcasted_iota(jnp.int32, t4.shape, 2)
    masked = jnp.where(li // nc == hi, t4, 0.0)
    s = jnp.sum(masked, axis=1)
    logits = jnp.dot(s, rfc_ref[...], preferred_element_type=jnp.float32)
    out_ref[...] = logits + bfc_ref[...]


def _toeplitz(w_oihw, e_of_dx, c_major=False):
    groups = []
    for dy in range(3):
        t = 0.0
        for dx in range(3):
            tap = w_oihw[:, :, dy, dx].T
            e = e_of_dx(dx)
            if c_major:
                blk = jnp.einsum('co,vw->cvwo', tap, e)
                blk = blk.reshape(tap.shape[0] * e.shape[0], -1)
            else:
                blk = jnp.einsum('co,vw->vcwo', tap, e)
                blk = blk.reshape(e.shape[0] * tap.shape[0], -1)
            t = t + blk
        groups.append(t)
    return jnp.concatenate(groups, axis=1).astype(_BF)


def _eye_sel(w, dx):
    return jnp.eye(w, k=1 - dx, dtype=jnp.float32)


def _pooled_sel(win, wout, dx):
    v = jnp.arange(win)[:, None]
    w = jnp.arange(wout)[None, :]
    return (v == 2 * (w + dx - 1)).astype(jnp.float32)


def kernel(x, w11, b11, w12, b12, w21, b21, w22, b22, wfc, bfc):
    N = x.shape[0]
    nc = wfc.shape[0]
    B = 32
    n_pad = (-N) % B

    x_in = x
    if n_pad:
        x_in = jnp.pad(x_in, ((0, n_pad), (0, 0), (0, 0), (0, 0)))
    Np = N + n_pad

    w1 = _toeplitz(w11, lambda dx: _eye_sel(32, dx), c_major=True)
    w2 = _toeplitz(w12, lambda dx: _eye_sel(32, dx))
    w3 = _toeplitz(w21, lambda dx: _pooled_sel(32, 16, dx))
    w4 = _toeplitz(w22, lambda dx: _eye_sel(16, dx))
    b1t = jnp.tile(b11, 32).reshape(1, 1024)
    b2t = jnp.tile(b12, 32).reshape(1, 1024)
    b3t = jnp.tile(b21, 16).reshape(1, 1024)
    b4t = jnp.tile(b22, 16).reshape(1, 1024)
    base = (wfc.reshape(nc, 64, 8, 8)
            .transpose(3, 1, 2, 0))
    wfc_k = (jnp.stack([base, jnp.zeros_like(base)], axis=1)
             .reshape(16 * 64, 8 * nc).astype(_BF))
    rfc = jnp.tile(jnp.eye(nc, dtype=jnp.float32), (8, 1))

    out = pl.pallas_call(
        _body,
        out_shape=jax.ShapeDtypeStruct((Np, nc), jnp.float32),
        grid=(Np // B,),
        in_specs=[
            pl.BlockSpec((B, 3, 32, 32), lambda n: (n, 0, 0, 0)),
            pl.BlockSpec((96, 3072), lambda n: (0, 0)),
            pl.BlockSpec((1, 1024), lambda n: (0, 0)),
            pl.BlockSpec((1024, 3072), lambda n: (0, 0)),
            pl.BlockSpec((1, 1024), lambda n: (0, 0)),
            pl.BlockSpec((1024, 3072), lambda n: (0, 0)),
            pl.BlockSpec((1, 1024), lambda n: (0, 0)),
            pl.BlockSpec((1024, 3072), lambda n: (0, 0)),
            pl.BlockSpec((1, 1024), lambda n: (0, 0)),
            pl.BlockSpec((1024, 8 * nc), lambda n: (0, 0)),
            pl.BlockSpec((8 * nc, nc), lambda n: (0, 0)),
            pl.BlockSpec((1, nc), lambda n: (0, 0)),
        ],
        out_specs=pl.BlockSpec((B, nc), lambda n: (n, 0)),
        compiler_params=pltpu.CompilerParams(
            dimension_semantics=("parallel",),
            vmem_limit_bytes=64 * 1024 * 1024,
        ),
    )(x_in, w1, b1t, w2, b2t, w3, b3t, w4, b4t, wfc_k, rfc,
      bfc.reshape(1, -1))
    return out[:N]
